# baseline (speedup 1.0000x reference)
import functools

import jax
import jax.numpy as jnp
from jax import lax
from jax.experimental import pallas as pl
from jax.experimental.pallas import tpu as pltpu
from jax.experimental.pallas import tpu_sc as plsc

_NUM_FIELDS = 26
_VOCAB = 100000
_EMBED_DIM = 32
_BATCH = 4096

_INFO = plsc.get_sparse_core_info()
_NC = _INFO.num_cores
_NS = _INFO.num_subcores
_NW = _NC * _NS
_N = _BATCH * _NUM_FIELDS
_NPW = _N // _NW
_CHUNK = 128
_NCHUNK = _NPW // _CHUNK
_VPC = _CHUNK // 16


def _make_gather():
    mesh = plsc.VectorSubcoreMesh(core_axis_name="c", subcore_axis_name="s")

    @functools.partial(
        pl.kernel,
        mesh=mesh,
        out_type=jax.ShapeDtypeStruct((_NW, _NCHUNK, _CHUNK, _EMBED_DIM),
                                      jnp.float32),
        scratch_types=[
            pltpu.VMEM((_NCHUNK, _CHUNK), jnp.int32),
            pltpu.VMEM((_NCHUNK, _CHUNK), jnp.int32),
            pltpu.VMEM((_NCHUNK, _CHUNK, _EMBED_DIM), jnp.float32),
            pltpu.SemaphoreType.DMA,
        ],
        compiler_params=pltpu.CompilerParams(use_tc_tiling_on_sc=False),
    )
    def gather_kernel(idx_hbm, offs_hbm, table_hbm, out_hbm,
                      idx_v, offs_v, rows_v, sem):
        wid = lax.axis_index("s") * _NC + lax.axis_index("c")
        pltpu.sync_copy(idx_hbm.at[wid], idx_v)
        pltpu.sync_copy(offs_hbm, offs_v)

        def fire(c, carry):
            for k in range(_VPC):
                sl = pl.ds(k * 16, 16)
                idx_v[c, sl] = idx_v[c, sl] + offs_v[c, sl]
            pltpu.async_copy(table_hbm.at[idx_v.at[c]], rows_v.at[c], sem)
            return carry

        lax.fori_loop(0, _NCHUNK, fire, 0)

        def drain(c, carry):
            pltpu.make_async_copy(table_hbm.at[idx_v.at[c]], rows_v.at[c],
                                  sem).wait()
            pltpu.sync_copy(rows_v.at[c], out_hbm.at[wid, c])
            return carry

        lax.fori_loop(0, _NCHUNK, drain, 0)

    return gather_kernel


_GATHER = _make_gather()


def kernel(inputs, tables):
    table_flat = tables.reshape(_NUM_FIELDS * _VOCAB, _EMBED_DIM)
    idx = inputs.astype(jnp.int32).reshape(_NW, _NCHUNK, _CHUNK)
    offs = jnp.tile(jnp.arange(_NUM_FIELDS, dtype=jnp.int32) * _VOCAB,
                    _NPW // _NUM_FIELDS).reshape(_NCHUNK, _CHUNK)
    out = _GATHER(idx, offs, table_flat)
    return out.reshape(_BATCH, 1, _NUM_FIELDS * _EMBED_DIM)
